# Initial kernel scaffold; baseline (speedup 1.0000x reference)
import jax
import jax.numpy as jnp
from jax.experimental import pallas as pl


def kernel(x, edge_index, nu, node_degrees, params):
    raise NotImplementedError("write your pallas kernel here")


# speedup vs baseline: 6.6696x; 6.6696x over previous
import functools

import jax
import jax.numpy as jnp
from jax import lax
from jax.experimental import pallas as pl
from jax.experimental.pallas import tpu as pltpu
from jax.experimental.pallas import tpu_sc as plsc

NC, NS = 2, 16
CHUNK = 128


def _mesh():
    return plsc.VectorSubcoreMesh(core_axis_name="c", subcore_axis_name="s")


def _make_sc_hist(N, E):
    per_tile = E // (NC * NS)
    nfull, tail = divmod(per_tile, CHUNK)
    rows_per_tile = N // NS

    @functools.partial(
        pl.kernel,
        out_type=jax.ShapeDtypeStruct((NC * N, 16), jnp.float32),
        mesh=_mesh(),
        compiler_params=pltpu.CompilerParams(use_tc_tiling_on_sc=False),
        scratch_types=[
            pltpu.VMEM((CHUNK,), jnp.int32),
            pltpu.VMEM((tail if tail else 8,), jnp.int32),
            pltpu.VMEM((CHUNK, 16), jnp.float32),
            pltpu.VMEM((rows_per_tile, 16), jnp.float32),
            pltpu.VMEM_SHARED((N, 16), jnp.float32),
        ],
    )
    def k(dst_hbm, zeros_hbm, out_hbm, didx, didx_t, ones, bounce, slab):
        cid = lax.axis_index("c")
        sid = lax.axis_index("s")
        wid = sid * NC + cid
        row0 = sid * rows_per_tile
        e0 = jnp.where(lax.iota(jnp.int32, 16) == 0,
                       jnp.float32(1.0), jnp.float32(0.0))

        def fill(i, c):
            ones[i, pl.ds(0, 16)] = e0
            return c
        lax.fori_loop(0, CHUNK, fill, 0)
        pltpu.sync_copy(zeros_hbm, slab.at[pl.ds(row0, rows_per_tile)])
        plsc.subcore_barrier()

        ebase = wid * per_tile

        def body(i, c):
            pltpu.sync_copy(dst_hbm.at[pl.ds(ebase + i * CHUNK, CHUNK)], didx)
            pltpu.sync_copy(ones, slab.at[didx], add=True)
            return c
        lax.fori_loop(0, nfull, body, 0)
        if tail:
            pltpu.sync_copy(dst_hbm.at[pl.ds(ebase + nfull * CHUNK, tail)], didx_t)
            pltpu.sync_copy(ones.at[pl.ds(0, tail)], slab.at[didx_t], add=True)
        plsc.subcore_barrier()
        pltpu.sync_copy(slab.at[pl.ds(row0, rows_per_tile)], bounce)
        pltpu.sync_copy(bounce, out_hbm.at[pl.ds(cid * N + row0, rows_per_tile)])

    return k


def _make_sc_segsum(N, E, W):
    per_tile = E // NS
    nfull, tail = divmod(per_tile, CHUNK)
    rows_per_tile = N // NS

    @functools.partial(
        pl.kernel,
        out_type=jax.ShapeDtypeStruct((NC * N, W), jnp.float32),
        mesh=_mesh(),
        compiler_params=pltpu.CompilerParams(use_tc_tiling_on_sc=False),
        scratch_types=[
            pltpu.VMEM((CHUNK,), jnp.int32),
            pltpu.VMEM((CHUNK,), jnp.int32),
            pltpu.VMEM((tail if tail else 8,), jnp.int32),
            pltpu.VMEM((tail if tail else 8,), jnp.int32),
            pltpu.VMEM((CHUNK, W), jnp.float32),
            pltpu.VMEM((tail if tail else 8, W), jnp.float32),
            pltpu.VMEM_SHARED((N, W), jnp.float32),
            pltpu.SemaphoreType.DMA,
        ],
    )
    def k(t0, t1, src_hbm, dst_hbm, zeros_hbm, out_hbm,
          sidx, didx, sidx_t, didx_t, rows, rows_t, slab, sem):
        cid = lax.axis_index("c")
        sid = lax.axis_index("s")
        row0 = sid * rows_per_tile
        pltpu.sync_copy(zeros_hbm, slab.at[pl.ds(row0, rows_per_tile)])
        plsc.subcore_barrier()

        ebase = sid * per_tile

        def body(i, c):
            b = ebase + i * CHUNK
            pltpu.sync_copy(src_hbm.at[pl.ds(b, CHUNK)], sidx)
            pltpu.sync_copy(dst_hbm.at[pl.ds(b, CHUNK)], didx)

            @pl.when(cid == 0)
            def _g0():
                pltpu.async_copy(t0.at[sidx], rows, sem).wait()

            @pl.when(cid == 1)
            def _g1():
                pltpu.async_copy(t1.at[sidx], rows, sem).wait()

            pltpu.sync_copy(rows, slab.at[didx], add=True)
            return c
        lax.fori_loop(0, nfull, body, 0)
        if tail:
            b = ebase + nfull * CHUNK
            pltpu.sync_copy(src_hbm.at[pl.ds(b, tail)], sidx_t)
            pltpu.sync_copy(dst_hbm.at[pl.ds(b, tail)], didx_t)

            @pl.when(cid == 0)
            def _t0():
                pltpu.async_copy(t0.at[sidx_t], rows_t, sem).wait()

            @pl.when(cid == 1)
            def _t1():
                pltpu.async_copy(t1.at[sidx_t], rows_t, sem).wait()

            pltpu.sync_copy(rows_t, slab.at[didx_t], add=True)
        plsc.subcore_barrier()
        pltpu.sync_copy(slab.at[pl.ds(row0, rows_per_tile)],
                        out_hbm.at[pl.ds(cid * N + row0, rows_per_tile)])

    return k


_BN = 2000


def _tc_pre(x, hist, nu2, fW1, fb1, fW2, fb2, aW1, ab1, aW2p, ab2p):
    N, D = x.shape
    grid = (N // _BN,)
    nb = N // _BN

    def body(x_r, h0_r, h1_r, nu_r, fW1_r, fb1_r, fW2_r, fb2_r,
             aW1_r, ab1_r, aW2_r, ab2_r,
             ls_o, filmed_o, hp0_o, hp1_o, dis_o):
        nu_t = nu_r[...]
        gb = jnp.maximum(nu_t @ fW1_r[...] + fb1_r[...], 0.0) @ fW2_r[...] + fb2_r[...]
        gamma, beta = gb[:, :D], gb[:, D:]
        aux_h = jnp.maximum(nu_t @ aW1_r[...] + ab1_r[...], 0.0)
        logits = aux_h @ aW2_r[...] + ab2_r[...]
        lane = lax.broadcasted_iota(jnp.int32, (1, 8), 1)
        logits = jnp.where(lane < 5, logits, -1e30)
        m = jnp.max(logits, axis=-1, keepdims=True)
        ew = jnp.exp(logits - m)
        aux_w = ew / jnp.sum(ew, axis=-1, keepdims=True)
        xv = x_r[...]
        ls_o[...] = jnp.sum(xv[:, :8] * aux_w, axis=1, keepdims=True)
        filmed = (1.0 + 0.5 * gamma) * xv + 0.3 * beta
        filmed_o[...] = filmed
        deg = h0_r[...][:, 0:1] + h1_r[...][:, 0:1] + 1.0
        dis = lax.rsqrt(deg)
        dis_o[...] = dis
        hp = filmed * dis
        hp0_o[...] = hp[:, :D // 2]
        hp1_o[...] = hp[:, D // 2:]

    full = lambda s: pl.BlockSpec(s, lambda i: (0, 0))
    outs = pl.pallas_call(
        body,
        grid=grid,
        in_specs=[
            pl.BlockSpec((_BN, D), lambda i: (i, 0)),
            pl.BlockSpec((_BN, 16), lambda i: (i, 0)),
            pl.BlockSpec((_BN, 16), lambda i, _nb=nb: (i + _nb, 0)),
            full((1, 1)),
            full(fW1.shape), full(fb1.shape), full(fW2.shape), full(fb2.shape),
            full(aW1.shape), full(ab1.shape), full(aW2p.shape), full(ab2p.shape),
        ],
        out_specs=[
            pl.BlockSpec((_BN, 1), lambda i: (i, 0)),
            pl.BlockSpec((_BN, D), lambda i: (i, 0)),
            pl.BlockSpec((_BN, D // 2), lambda i: (i, 0)),
            pl.BlockSpec((_BN, D // 2), lambda i: (i, 0)),
            pl.BlockSpec((_BN, 1), lambda i: (i, 0)),
        ],
        out_shape=[
            jax.ShapeDtypeStruct((N, 1), jnp.float32),
            jax.ShapeDtypeStruct((N, D), jnp.float32),
            jax.ShapeDtypeStruct((N, D // 2), jnp.float32),
            jax.ShapeDtypeStruct((N, D // 2), jnp.float32),
            jax.ShapeDtypeStruct((N, 1), jnp.float32),
        ],
    )(x, hist, hist, nu2, fW1, fb1, fW2, fb2, aW1, ab1, aW2p, ab2p)
    return outs


def _tc_layer(slab, h, dis, W, b, relu):
    N, K = h.shape
    H2 = W.shape[1]
    nb = N // _BN

    def body(s0_r, s1_r, h_r, dis_r, W_r, b_r, hn_o, hp0_o, hp1_o):
        dis = dis_r[...]
        di = dis * dis
        agg = jnp.concatenate([s0_r[...], s1_r[...]], axis=1) * dis + h_r[...] * di
        z = jnp.dot(agg, W_r[...], preferred_element_type=jnp.float32) + b_r[...]
        if relu:
            z = jnp.maximum(z, 0.0)
        hn_o[...] = z
        hp = z * dis
        hp0_o[...] = hp[:, :H2 // 2]
        hp1_o[...] = hp[:, H2 // 2:]

    return pl.pallas_call(
        body,
        grid=(nb,),
        in_specs=[
            pl.BlockSpec((_BN, K // 2), lambda i: (i, 0)),
            pl.BlockSpec((_BN, K // 2), lambda i, _nb=nb: (i + _nb, 0)),
            pl.BlockSpec((_BN, K), lambda i: (i, 0)),
            pl.BlockSpec((_BN, 1), lambda i: (i, 0)),
            pl.BlockSpec(W.shape, lambda i: (0, 0)),
            pl.BlockSpec(b.shape, lambda i: (0, 0)),
        ],
        out_specs=[
            pl.BlockSpec((_BN, H2), lambda i: (i, 0)),
            pl.BlockSpec((_BN, H2 // 2), lambda i: (i, 0)),
            pl.BlockSpec((_BN, H2 // 2), lambda i: (i, 0)),
        ],
        out_shape=[
            jax.ShapeDtypeStruct((N, H2), jnp.float32),
            jax.ShapeDtypeStruct((N, H2 // 2), jnp.float32),
            jax.ShapeDtypeStruct((N, H2 // 2), jnp.float32),
        ],
    )(slab, slab, h, dis, W, b)


def _tc_final(slab, h, dis, ndeg, nu2, W3, b3, attW1h, attW1nu, attW1dg, attb1,
              attW2, attb2, outW1, outb1, outW2, outb2):
    N, K = h.shape
    nb = N // _BN

    def body(s0_r, s1_r, h_r, dis_r, nd_r, nu_r, W3_r, b3_r, aW1_r, aWn_r,
             aWd_r, ab1_r, aW2_r, ab2_r, oW1_r, ob1_r, oW2_r, ob2_r, main_o):
        dis = dis_r[...]
        di = dis * dis
        agg = jnp.concatenate([s0_r[...], s1_r[...]], axis=1) * dis + h_r[...] * di
        h4 = jnp.dot(agg, W3_r[...], preferred_element_type=jnp.float32) + b3_r[...]
        t = (jnp.dot(h4, aW1_r[...], preferred_element_type=jnp.float32)
             + nu_r[...] * aWn_r[...] + nd_r[...] * aWd_r[...] + ab1_r[...])
        t = jnp.maximum(t, 0.0)
        aw = jnp.dot(t, aW2_r[...], preferred_element_type=jnp.float32) + ab2_r[...]
        aw = 1.0 / (1.0 + jnp.exp(-aw))
        att = h4 * aw
        u = jnp.maximum(
            jnp.dot(att, oW1_r[...], preferred_element_type=jnp.float32) + ob1_r[...], 0.0)
        main_o[...] = jnp.dot(u, oW2_r[...], preferred_element_type=jnp.float32) + ob2_r[...]

    full = lambda s: pl.BlockSpec(s, lambda i: (0, 0))
    return pl.pallas_call(
        body,
        grid=(nb,),
        in_specs=[
            pl.BlockSpec((_BN, K // 2), lambda i: (i, 0)),
            pl.BlockSpec((_BN, K // 2), lambda i, _nb=nb: (i + _nb, 0)),
            pl.BlockSpec((_BN, K), lambda i: (i, 0)),
            pl.BlockSpec((_BN, 1), lambda i: (i, 0)),
            pl.BlockSpec((_BN, 1), lambda i: (i, 0)),
            full((1, 1)),
            full(W3.shape), full(b3.shape),
            full(attW1h.shape), full(attW1nu.shape), full(attW1dg.shape),
            full(attb1.shape), full(attW2.shape), full(attb2.shape),
            full(outW1.shape), full(outb1.shape), full(outW2.shape), full(outb2.shape),
        ],
        out_specs=[pl.BlockSpec((_BN, 1), lambda i: (i, 0))],
        out_shape=[jax.ShapeDtypeStruct((N, 1), jnp.float32)],
    )(slab, slab, h, dis, ndeg, nu2, W3, b3, attW1h, attW1nu, attW1dg, attb1,
      attW2, attb2, outW1, outb1, outW2, outb2)[0]


def kernel(x, edge_index, nu, node_degrees, params):
    p = params
    N, D = x.shape
    E = edge_index.shape[1]
    H = p["gcn_W1"].shape[1]
    src = edge_index[0]
    dst = edge_index[1]
    nu2 = nu.reshape(1, 1)
    rows_per_tile = N // NS
    zeros_w = jnp.zeros((rows_per_tile, max(H // 2, 16)), jnp.float32)

    hist = _make_sc_hist(N, E)(dst, zeros_w[:, :16])

    aW2p = jnp.pad(p["aux_W2"], ((0, 0), (0, 3)))
    ab2p = jnp.pad(p["aux_b2"], (0, 3)).reshape(1, 8)
    ls, filmed, hp0, hp1, dis = _tc_pre(
        x, hist, nu2,
        p["film_W1"], p["film_b1"].reshape(1, -1),
        p["film_W2"], p["film_b2"].reshape(1, -1),
        p["aux_W1"], p["aux_b1"].reshape(1, -1), aW2p, ab2p)

    s1 = _make_sc_segsum(N, E, D // 2)(hp0, hp1, src, dst, zeros_w[:, :D // 2])
    h2, h2p0, h2p1 = _tc_layer(s1, filmed, dis, p["gcn_W1"],
                               p["gcn_b1"].reshape(1, -1), True)
    s2 = _make_sc_segsum(N, E, H // 2)(h2p0, h2p1, src, dst, zeros_w[:, :H // 2])
    h3, h3p0, h3p1 = _tc_layer(s2, h2, dis, p["gcn_W2"],
                               p["gcn_b2"].reshape(1, -1), True)
    s3 = _make_sc_segsum(N, E, H // 2)(h3p0, h3p1, src, dst, zeros_w[:, :H // 2])

    main = _tc_final(
        s3, h3, dis, node_degrees, nu2,
        p["gcn_W3"], p["gcn_b3"].reshape(1, -1),
        p["att_W1"][:H], p["att_W1"][H:H + 1], p["att_W1"][H + 1:H + 2],
        p["att_b1"].reshape(1, -1), p["att_W2"], p["att_b2"].reshape(1, -1),
        p["out_W1"], p["out_b1"].reshape(1, -1),
        p["out_W2"], p["out_b2"].reshape(1, -1))
    return main, ls


# speedup vs baseline: 11.8855x; 1.7820x over previous
import functools

import jax
import jax.numpy as jnp
from jax import lax
from jax.experimental import pallas as pl
from jax.experimental.pallas import tpu as pltpu
from jax.experimental.pallas import tpu_sc as plsc

NC, NS = 2, 16
CHUNK = 128


def _mesh():
    return plsc.VectorSubcoreMesh(core_axis_name="c", subcore_axis_name="s")


def _make_sc_hist(N, E):
    per_tile = E // (NC * NS)
    nfull, tail = divmod(per_tile, CHUNK)
    rows_per_tile = N // NS

    @functools.partial(
        pl.kernel,
        out_type=jax.ShapeDtypeStruct((NC * N, 16), jnp.float32),
        mesh=_mesh(),
        compiler_params=pltpu.CompilerParams(use_tc_tiling_on_sc=False),
        scratch_types=[
            pltpu.VMEM((CHUNK,), jnp.int32),
            pltpu.VMEM((tail if tail else 8,), jnp.int32),
            pltpu.VMEM((CHUNK, 16), jnp.float32),
            pltpu.VMEM((rows_per_tile, 16), jnp.float32),
            pltpu.VMEM_SHARED((N, 16), jnp.float32),
        ],
    )
    def k(dst_hbm, zeros_hbm, out_hbm, didx, didx_t, ones, bounce, slab):
        cid = lax.axis_index("c")
        sid = lax.axis_index("s")
        wid = sid * NC + cid
        row0 = sid * rows_per_tile
        e0 = jnp.where(lax.iota(jnp.int32, 16) == 0,
                       jnp.float32(1.0), jnp.float32(0.0))

        def fill(i, c):
            ones[i, pl.ds(0, 16)] = e0
            return c
        lax.fori_loop(0, CHUNK, fill, 0)
        pltpu.sync_copy(zeros_hbm, slab.at[pl.ds(row0, rows_per_tile)])
        plsc.subcore_barrier()

        ebase = wid * per_tile

        def body(i, c):
            pltpu.sync_copy(dst_hbm.at[pl.ds(ebase + i * CHUNK, CHUNK)], didx)
            pltpu.sync_copy(ones, slab.at[didx], add=True)
            return c
        lax.fori_loop(0, nfull, body, 0)
        if tail:
            pltpu.sync_copy(dst_hbm.at[pl.ds(ebase + nfull * CHUNK, tail)], didx_t)
            pltpu.sync_copy(ones.at[pl.ds(0, tail)], slab.at[didx_t], add=True)
        plsc.subcore_barrier()
        pltpu.sync_copy(slab.at[pl.ds(row0, rows_per_tile)], bounce)
        pltpu.sync_copy(bounce, out_hbm.at[pl.ds(cid * N + row0, rows_per_tile)])

    return k


def _make_sc_segsum(N, E, W):
    per_tile = E // NS
    nfull, tail = divmod(per_tile, CHUNK)
    rows_per_tile = N // NS
    assert nfull >= 4 and (nfull - 2) % 2 == 0 and tail % 8 == 0

    @functools.partial(
        pl.kernel,
        out_type=jax.ShapeDtypeStruct((NC * N, W), jnp.float32),
        mesh=_mesh(),
        compiler_params=pltpu.CompilerParams(use_tc_tiling_on_sc=False),
        scratch_types=[
            pltpu.VMEM((CHUNK,), jnp.int32),
            pltpu.VMEM((CHUNK,), jnp.int32),
            pltpu.VMEM((CHUNK,), jnp.int32),
            pltpu.VMEM((CHUNK,), jnp.int32),
            pltpu.VMEM((tail if tail else 8,), jnp.int32),
            pltpu.VMEM((tail if tail else 8,), jnp.int32),
            pltpu.VMEM((CHUNK, W), jnp.float32),
            pltpu.VMEM((CHUNK, W), jnp.float32),
            pltpu.VMEM((tail if tail else 8, W), jnp.float32),
            pltpu.VMEM_SHARED((N, W), jnp.float32),
            pltpu.SemaphoreType.DMA,
            pltpu.SemaphoreType.DMA,
            pltpu.SemaphoreType.DMA,
            pltpu.SemaphoreType.DMA,
            pltpu.SemaphoreType.DMA,
        ],
    )
    def k(t0, t1, src_hbm, dst_hbm, zeros_hbm, out_hbm,
          sidx0, sidx1, didx0, didx1, sidx_t, didx_t, rows0, rows1, rows_t,
          slab, gsem0, gsem1, isem0, isem1, tsem):
        cid = lax.axis_index("c")
        sid = lax.axis_index("s")
        row0 = sid * rows_per_tile
        sidx = (sidx0, sidx1)
        didx = (didx0, didx1)
        rows = (rows0, rows1)
        gsem = (gsem0, gsem1)
        isem = (isem0, isem1)
        pltpu.sync_copy(zeros_hbm, slab.at[pl.ds(row0, rows_per_tile)])
        plsc.subcore_barrier()

        ebase = sid * per_tile

        def issue_idx(i, b):
            off = ebase + i * CHUNK
            pltpu.async_copy(src_hbm.at[pl.ds(off, CHUNK)], sidx[b], isem[b])
            pltpu.async_copy(dst_hbm.at[pl.ds(off, CHUNK)], didx[b], isem[b])

        def wait_idx(i, b):
            off = ebase + i * CHUNK
            pltpu.make_async_copy(src_hbm.at[pl.ds(off, CHUNK)], sidx[b], isem[b]).wait()
            pltpu.make_async_copy(dst_hbm.at[pl.ds(off, CHUNK)], didx[b], isem[b]).wait()

        def issue_gather(b):
            @pl.when(cid == 0)
            def _g0():
                pltpu.async_copy(t0.at[sidx[b]], rows[b], gsem[b])

            @pl.when(cid == 1)
            def _g1():
                pltpu.async_copy(t1.at[sidx[b]], rows[b], gsem[b])

        def wait_gather(b):
            @pl.when(cid == 0)
            def _w0():
                pltpu.make_async_copy(t0.at[sidx[b]], rows[b], gsem[b]).wait()

            @pl.when(cid == 1)
            def _w1():
                pltpu.make_async_copy(t1.at[sidx[b]], rows[b], gsem[b]).wait()

        issue_idx(0, 0)
        wait_idx(0, 0)
        issue_gather(0)
        issue_idx(1, 1)

        def chunk_body(i, b, prefetch):
            wait_gather(b)
            wait_idx(i + 1, 1 - b)
            issue_gather(1 - b)
            pltpu.sync_copy(rows[b], slab.at[didx[b]], add=True)
            if prefetch:
                issue_idx(i + 2, b)

        def body2(s, c):
            i = s * 2
            chunk_body(i, 0, True)
            chunk_body(i + 1, 1, True)
            return c
        lax.fori_loop(0, (nfull - 2) // 2, body2, 0)
        chunk_body(nfull - 2, 0, False)
        wait_gather(1)
        pltpu.sync_copy(rows[1], slab.at[didx[1]], add=True)
        if tail:
            b = ebase + nfull * CHUNK
            pltpu.sync_copy(src_hbm.at[pl.ds(b, tail)], sidx_t)
            pltpu.sync_copy(dst_hbm.at[pl.ds(b, tail)], didx_t)

            @pl.when(cid == 0)
            def _t0():
                pltpu.async_copy(t0.at[sidx_t], rows_t, tsem).wait()

            @pl.when(cid == 1)
            def _t1():
                pltpu.async_copy(t1.at[sidx_t], rows_t, tsem).wait()

            pltpu.sync_copy(rows_t, slab.at[didx_t], add=True)
        plsc.subcore_barrier()
        pltpu.sync_copy(slab.at[pl.ds(row0, rows_per_tile)],
                        out_hbm.at[pl.ds(cid * N + row0, rows_per_tile)])

    return k


_BN = 2000


def _tc_pre(x, hist, nu2, fW1, fb1, fW2, fb2, aW1, ab1, aW2p, ab2p):
    N, D = x.shape
    grid = (N // _BN,)
    nb = N // _BN

    def body(x_r, h0_r, h1_r, nu_r, fW1_r, fb1_r, fW2_r, fb2_r,
             aW1_r, ab1_r, aW2_r, ab2_r,
             ls_o, filmed_o, hp0_o, hp1_o, dis_o):
        nu_t = nu_r[...]
        gb = jnp.maximum(nu_t @ fW1_r[...] + fb1_r[...], 0.0) @ fW2_r[...] + fb2_r[...]
        gamma, beta = gb[:, :D], gb[:, D:]
        aux_h = jnp.maximum(nu_t @ aW1_r[...] + ab1_r[...], 0.0)
        logits = aux_h @ aW2_r[...] + ab2_r[...]
        lane = lax.broadcasted_iota(jnp.int32, (1, 8), 1)
        logits = jnp.where(lane < 5, logits, -1e30)
        m = jnp.max(logits, axis=-1, keepdims=True)
        ew = jnp.exp(logits - m)
        aux_w = ew / jnp.sum(ew, axis=-1, keepdims=True)
        xv = x_r[...]
        ls_o[...] = jnp.sum(xv[:, :8] * aux_w, axis=1, keepdims=True)
        filmed = (1.0 + 0.5 * gamma) * xv + 0.3 * beta
        filmed_o[...] = filmed
        deg = h0_r[...][:, 0:1] + h1_r[...][:, 0:1] + 1.0
        dis = lax.rsqrt(deg)
        dis_o[...] = dis
        hp = filmed * dis
        hp0_o[...] = hp[:, :D // 2]
        hp1_o[...] = hp[:, D // 2:]

    full = lambda s: pl.BlockSpec(s, lambda i: (0, 0))
    outs = pl.pallas_call(
        body,
        grid=grid,
        in_specs=[
            pl.BlockSpec((_BN, D), lambda i: (i, 0)),
            pl.BlockSpec((_BN, 16), lambda i: (i, 0)),
            pl.BlockSpec((_BN, 16), lambda i, _nb=nb: (i + _nb, 0)),
            full((1, 1)),
            full(fW1.shape), full(fb1.shape), full(fW2.shape), full(fb2.shape),
            full(aW1.shape), full(ab1.shape), full(aW2p.shape), full(ab2p.shape),
        ],
        out_specs=[
            pl.BlockSpec((_BN, 1), lambda i: (i, 0)),
            pl.BlockSpec((_BN, D), lambda i: (i, 0)),
            pl.BlockSpec((_BN, D // 2), lambda i: (i, 0)),
            pl.BlockSpec((_BN, D // 2), lambda i: (i, 0)),
            pl.BlockSpec((_BN, 1), lambda i: (i, 0)),
        ],
        out_shape=[
            jax.ShapeDtypeStruct((N, 1), jnp.float32),
            jax.ShapeDtypeStruct((N, D), jnp.float32),
            jax.ShapeDtypeStruct((N, D // 2), jnp.float32),
            jax.ShapeDtypeStruct((N, D // 2), jnp.float32),
            jax.ShapeDtypeStruct((N, 1), jnp.float32),
        ],
    )(x, hist, hist, nu2, fW1, fb1, fW2, fb2, aW1, ab1, aW2p, ab2p)
    return outs


def _tc_layer(slab, h, dis, W, b, relu):
    N, K = h.shape
    H2 = W.shape[1]
    nb = N // _BN

    def body(s0_r, s1_r, h_r, dis_r, W_r, b_r, hn_o, hp0_o, hp1_o):
        dis = dis_r[...]
        di = dis * dis
        agg = jnp.concatenate([s0_r[...], s1_r[...]], axis=1) * dis + h_r[...] * di
        z = jnp.dot(agg, W_r[...], preferred_element_type=jnp.float32) + b_r[...]
        if relu:
            z = jnp.maximum(z, 0.0)
        hn_o[...] = z
        hp = z * dis
        hp0_o[...] = hp[:, :H2 // 2]
        hp1_o[...] = hp[:, H2 // 2:]

    return pl.pallas_call(
        body,
        grid=(nb,),
        in_specs=[
            pl.BlockSpec((_BN, K // 2), lambda i: (i, 0)),
            pl.BlockSpec((_BN, K // 2), lambda i, _nb=nb: (i + _nb, 0)),
            pl.BlockSpec((_BN, K), lambda i: (i, 0)),
            pl.BlockSpec((_BN, 1), lambda i: (i, 0)),
            pl.BlockSpec(W.shape, lambda i: (0, 0)),
            pl.BlockSpec(b.shape, lambda i: (0, 0)),
        ],
        out_specs=[
            pl.BlockSpec((_BN, H2), lambda i: (i, 0)),
            pl.BlockSpec((_BN, H2 // 2), lambda i: (i, 0)),
            pl.BlockSpec((_BN, H2 // 2), lambda i: (i, 0)),
        ],
        out_shape=[
            jax.ShapeDtypeStruct((N, H2), jnp.float32),
            jax.ShapeDtypeStruct((N, H2 // 2), jnp.float32),
            jax.ShapeDtypeStruct((N, H2 // 2), jnp.float32),
        ],
    )(slab, slab, h, dis, W, b)


def _tc_final(slab, h, dis, ndeg, nu2, W3, b3, attW1h, attW1nu, attW1dg, attb1,
              attW2, attb2, outW1, outb1, outW2, outb2):
    N, K = h.shape
    nb = N // _BN

    def body(s0_r, s1_r, h_r, dis_r, nd_r, nu_r, W3_r, b3_r, aW1_r, aWn_r,
             aWd_r, ab1_r, aW2_r, ab2_r, oW1_r, ob1_r, oW2_r, ob2_r, main_o):
        dis = dis_r[...]
        di = dis * dis
        agg = jnp.concatenate([s0_r[...], s1_r[...]], axis=1) * dis + h_r[...] * di
        h4 = jnp.dot(agg, W3_r[...], preferred_element_type=jnp.float32) + b3_r[...]
        t = (jnp.dot(h4, aW1_r[...], preferred_element_type=jnp.float32)
             + nu_r[...] * aWn_r[...] + nd_r[...] * aWd_r[...] + ab1_r[...])
        t = jnp.maximum(t, 0.0)
        aw = jnp.dot(t, aW2_r[...], preferred_element_type=jnp.float32) + ab2_r[...]
        aw = 1.0 / (1.0 + jnp.exp(-aw))
        att = h4 * aw
        u = jnp.maximum(
            jnp.dot(att, oW1_r[...], preferred_element_type=jnp.float32) + ob1_r[...], 0.0)
        main_o[...] = jnp.dot(u, oW2_r[...], preferred_element_type=jnp.float32) + ob2_r[...]

    full = lambda s: pl.BlockSpec(s, lambda i: (0, 0))
    return pl.pallas_call(
        body,
        grid=(nb,),
        in_specs=[
            pl.BlockSpec((_BN, K // 2), lambda i: (i, 0)),
            pl.BlockSpec((_BN, K // 2), lambda i, _nb=nb: (i + _nb, 0)),
            pl.BlockSpec((_BN, K), lambda i: (i, 0)),
            pl.BlockSpec((_BN, 1), lambda i: (i, 0)),
            pl.BlockSpec((_BN, 1), lambda i: (i, 0)),
            full((1, 1)),
            full(W3.shape), full(b3.shape),
            full(attW1h.shape), full(attW1nu.shape), full(attW1dg.shape),
            full(attb1.shape), full(attW2.shape), full(attb2.shape),
            full(outW1.shape), full(outb1.shape), full(outW2.shape), full(outb2.shape),
        ],
        out_specs=[pl.BlockSpec((_BN, 1), lambda i: (i, 0))],
        out_shape=[jax.ShapeDtypeStruct((N, 1), jnp.float32)],
    )(slab, slab, h, dis, ndeg, nu2, W3, b3, attW1h, attW1nu, attW1dg, attb1,
      attW2, attb2, outW1, outb1, outW2, outb2)[0]


def kernel(x, edge_index, nu, node_degrees, params):
    p = params
    N, D = x.shape
    E = edge_index.shape[1]
    H = p["gcn_W1"].shape[1]
    src = edge_index[0]
    dst = edge_index[1]
    nu2 = nu.reshape(1, 1)
    rows_per_tile = N // NS
    zeros_w = jnp.zeros((rows_per_tile, max(H // 2, 16)), jnp.float32)

    hist = _make_sc_hist(N, E)(dst, zeros_w[:, :16])

    aW2p = jnp.pad(p["aux_W2"], ((0, 0), (0, 3)))
    ab2p = jnp.pad(p["aux_b2"], (0, 3)).reshape(1, 8)
    ls, filmed, hp0, hp1, dis = _tc_pre(
        x, hist, nu2,
        p["film_W1"], p["film_b1"].reshape(1, -1),
        p["film_W2"], p["film_b2"].reshape(1, -1),
        p["aux_W1"], p["aux_b1"].reshape(1, -1), aW2p, ab2p)

    s1 = _make_sc_segsum(N, E, D // 2)(hp0, hp1, src, dst, zeros_w[:, :D // 2])
    h2, h2p0, h2p1 = _tc_layer(s1, filmed, dis, p["gcn_W1"],
                               p["gcn_b1"].reshape(1, -1), True)
    s2 = _make_sc_segsum(N, E, H // 2)(h2p0, h2p1, src, dst, zeros_w[:, :H // 2])
    h3, h3p0, h3p1 = _tc_layer(s2, h2, dis, p["gcn_W2"],
                               p["gcn_b2"].reshape(1, -1), True)
    s3 = _make_sc_segsum(N, E, H // 2)(h3p0, h3p1, src, dst, zeros_w[:, :H // 2])

    main = _tc_final(
        s3, h3, dis, node_degrees, nu2,
        p["gcn_W3"], p["gcn_b3"].reshape(1, -1),
        p["att_W1"][:H], p["att_W1"][H:H + 1], p["att_W1"][H + 1:H + 2],
        p["att_b1"].reshape(1, -1), p["att_W2"], p["att_b2"].reshape(1, -1),
        p["out_W1"], p["out_b1"].reshape(1, -1),
        p["out_W2"], p["out_b2"].reshape(1, -1))
    return main, ls
